# baseline (speedup 1.0000x reference)
import functools

import jax
import jax.numpy as jnp
from jax import lax
from jax.experimental import pallas as pl
from jax.experimental.pallas import tpu as pltpu
from jax.experimental.pallas import tpu_sc as plsc

N_NODES = 100000
N_EDGES = 1600000
RS = 6.0
INV_W = 5.0
BOHR = 1.8897259886
LN2 = 0.6931471805599453

NW = 32
CE = 3200
NT = CE // 128
NCH = N_EDGES // CE
SMAX = -(-NCH // NW)
NODE_WORKERS = 25
NPW = N_NODES // NODE_WORKERS

_REFLECTIVE = (0, 2, 1, 3)
_SAME_ORB = (0, 3, 4, 7, 8, 9)

_i32 = jnp.int32
_f32 = jnp.float32


def _sc_body(eat, el, at, tab, edge_out, node_out,
             eat_v0, eat_v1, el_v0, el_v1, out_a0, out_a1, out_b0, out_b1,
             at_v, outn_v, tab_v, in_s0, in_s1, out_s0, out_s1):
    eat_vs = (eat_v0, eat_v1)
    el_vs = (el_v0, el_v1)
    out_as = (out_a0, out_a1)
    out_bs = (out_b0, out_b1)
    in_sems = (in_s0, in_s1)
    out_sems = (out_s0, out_s1)

    wid = lax.axis_index("s") * 2 + lax.axis_index("c")
    pltpu.sync_copy(tab, tab_v)
    lane = lax.iota(_i32, 16)

    def start_in(c, b):
        base = c * CE
        pltpu.async_copy(eat.at[:, pl.ds(base, CE)], eat_vs[b], in_sems[b])
        pltpu.async_copy(el.at[pl.ds(base, CE)], el_vs[b], in_sems[b])

    def wait_in(b):
        pltpu.make_async_copy(
            eat.at[:, pl.ds(0, CE)], eat_vs[b], in_sems[b]).wait()
        pltpu.make_async_copy(
            el.at[pl.ds(0, CE)], el_vs[b], in_sems[b]).wait()

    def start_out(c, b):
        tb = c * NT
        pltpu.async_copy(
            out_as[b], edge_out.at[0, pl.ds(tb, NT), :, :], out_sems[b])
        pltpu.async_copy(
            out_bs[b], edge_out.at[1, pl.ds(tb, NT), pl.ds(0, 2), :],
            out_sems[b])

    def wait_out(b):
        pltpu.make_async_copy(
            out_as[b], edge_out.at[0, pl.ds(0, NT), :, :], out_sems[b]).wait()
        pltpu.make_async_copy(
            out_bs[b], edge_out.at[1, pl.ds(0, NT), pl.ds(0, 2), :],
            out_sems[b]).wait()

    def compute(b):
        eat_v, el_v, out_a, out_b = eat_vs[b], el_vs[b], out_as[b], out_bs[b]

        @plsc.parallel_loop(0, CE // 16, unroll=6)
        def _(i):
            t = i >> 3
            sub = (i & 7) * 16
            sl = pl.ds(i * 16, 16)
            ti = eat_v[0, sl]
            tj = eat_v[1, sl]
            r = el_v[sl]
            idx = ti * 2 + tj
            bits = lax.bitcast_convert_type(r, _i32)
            ef = ((bits >> 23) - 127).astype(_f32)
            m = lax.bitcast_convert_type(
                (bits & 0x007FFFFF) | 0x3F800000, _f32)
            sv = (m - 1.0) / (m + 1.0)
            s2 = sv * sv
            lnr = ef * LN2 + (2.0 * sv) * (1.0 + s2 * (1.0 / 3.0 + s2 * 0.2))
            for_ = 1.0 / (r * (1.0 + jnp.exp((r - RS) * INV_W)))
            dsub = pl.ds(sub, 16)
            for k in range(10):
                w = plsc.load_gather(tab_v, [idx + (4 * k)])
                cval = lax.bitcast_convert_type(w & -65536, _f32)
                na2 = lax.bitcast_convert_type(w << 16, _f32)
                v = cval * jnp.exp(na2 * lnr) * for_
                if k < 8:
                    out_a[t, k, dsub] = v
                else:
                    out_b[t, k - 8, dsub] = v

    start_in(wid, 0)

    def pair_body(p, carry):
        for h in range(2):
            s = 2 * p + h
            c = wid + s * NW

            @pl.when(c < NCH)
            def _(c=c, h=h, s=s, p=p):
                cn = wid + (s + 1) * NW

                @pl.when(cn < NCH)
                def _():
                    start_in(cn, 1 - h)

                wait_in(h)

                @pl.when(p > 0)
                def _():
                    wait_out(h)

                compute(h)
                start_out(c, h)

        return carry

    lax.fori_loop(0, SMAX // 2, pair_body, 0)
    wait_out(0)
    wait_out(1)

    @pl.when(wid < NODE_WORKERS)
    def _():
        nbase = wid * NPW
        pltpu.sync_copy(at.at[pl.ds(nbase, NPW)], at_v)

        @plsc.parallel_loop(0, NPW // 16, unroll=4)
        def _(i):
            a = at_v[pl.ds(i * 16, 16)]
            nids = lane + i * 16
            for j in range(3):
                w = plsc.load_gather(tab_v, [a + (40 + 2 * j)])
                plsc.store_scatter(
                    outn_v, [nids, jnp.full((16,), j, _i32)],
                    lax.bitcast_convert_type(w, _f32))

        pltpu.sync_copy(outn_v, node_out.at[pl.ds(nbase, NPW), :])


_sc_kernel = functools.partial(
    pl.kernel,
    _sc_body,
    out_type=(
        jax.ShapeDtypeStruct((2, N_EDGES // 128, 8, 128), jnp.float32),
        jax.ShapeDtypeStruct((N_NODES, 3), jnp.float32),
    ),
    mesh=plsc.VectorSubcoreMesh(core_axis_name="c", subcore_axis_name="s"),
    compiler_params=pltpu.CompilerParams(
        needs_layout_passes=False, use_tc_tiling_on_sc=False),
    scratch_types=[
        pltpu.VMEM((2, CE), jnp.int32),
        pltpu.VMEM((2, CE), jnp.int32),
        pltpu.VMEM((CE,), jnp.float32),
        pltpu.VMEM((CE,), jnp.float32),
        pltpu.VMEM((NT, 8, 128), jnp.float32),
        pltpu.VMEM((NT, 8, 128), jnp.float32),
        pltpu.VMEM((NT, 2, 128), jnp.float32),
        pltpu.VMEM((NT, 2, 128), jnp.float32),
        pltpu.VMEM((NPW,), jnp.int32),
        pltpu.VMEM((NPW, 3), jnp.float32),
        pltpu.VMEM((64,), jnp.int32),
        pltpu.SemaphoreType.DMA,
        pltpu.SemaphoreType.DMA,
        pltpu.SemaphoreType.DMA,
        pltpu.SemaphoreType.DMA,
    ],
)()


def kernel(edge_atom_types, edge_length, atom_type, hopping_param,
           onsite_param, bond_length_table):
    refl = hopping_param[jnp.array(_REFLECTIVE, dtype=_i32)]
    mask = jnp.zeros((10,), dtype=hopping_param.dtype)
    mask = mask.at[jnp.array(_SAME_ORB, dtype=_i32)].set(1.0)[None, :, None]
    hp = hopping_param * (1.0 - mask) + 0.5 * (hopping_param + refl) * mask

    alpha1 = hp[..., 0]
    a2 = jnp.abs(hp[..., 1])
    ti_b = jnp.array([0, 0, 1, 1], dtype=_i32)
    tj_b = jnp.array([0, 1, 0, 1], dtype=_i32)
    r0b = 0.5 * (bond_length_table[ti_b] + bond_length_table[tj_b]) / BOHR
    cb = alpha1 * r0b[:, None] ** (1.0 + a2)
    node_t = onsite_param[:, :, 0]

    cb_bits = lax.bitcast_convert_type(
        cb.astype(jnp.bfloat16), jnp.uint16).astype(_i32)
    na2_bits = lax.bitcast_convert_type(
        (-a2).astype(jnp.bfloat16), jnp.uint16).astype(_i32)
    packed = (cb_bits << 16) | na2_bits

    tab = jnp.zeros((64,), dtype=_i32)
    tab = tab.at[0:40].set(packed.T.reshape(-1))
    tab = tab.at[40:46].set(lax.bitcast_convert_type(
        node_t.astype(_f32), _i32).T.reshape(-1))
    edge_tiles, node_features = _sc_kernel(
        edge_atom_types.astype(_i32), edge_length,
        atom_type.astype(_i32), tab)
    edge_features = edge_tiles.transpose(1, 3, 0, 2).reshape(
        N_EDGES, 16)[:, :10]
    return edge_features, node_features


# speedup vs baseline: 1.0463x; 1.0463x over previous
import functools

import jax
import jax.numpy as jnp
from jax import lax
from jax.experimental import pallas as pl
from jax.experimental.pallas import tpu as pltpu
from jax.experimental.pallas import tpu_sc as plsc

N_NODES = 100000
N_EDGES = 1600000
RS = 6.0
INV_W = 5.0
BOHR = 1.8897259886
LN2 = 0.6931471805599453

NW = 32
CE = 3200
NT = CE // 128
NCH = N_EDGES // CE
SMAX = -(-NCH // NW)
NODE_WORKERS = 25
NPW = N_NODES // NODE_WORKERS

_REFLECTIVE = (0, 2, 1, 3)
_SAME_ORB = (0, 3, 4, 7, 8, 9)

_i32 = jnp.int32
_f32 = jnp.float32


def _sc_body(eat, el, at, tab, edge_out, node_out,
             eat_v0, eat_v1, el_v0, el_v1, out_a0, out_a1, out_b0, out_b1,
             at_v, outn_v, tab_v, in_s0, in_s1, out_s0, out_s1):
    eat_vs = (eat_v0, eat_v1)
    el_vs = (el_v0, el_v1)
    out_as = (out_a0, out_a1)
    out_bs = (out_b0, out_b1)
    in_sems = (in_s0, in_s1)
    out_sems = (out_s0, out_s1)

    wid = lax.axis_index("s") * 2 + lax.axis_index("c")
    pltpu.sync_copy(tab, tab_v)
    lane = lax.iota(_i32, 16)

    def start_in(c, b):
        base = c * CE
        pltpu.async_copy(eat.at[:, pl.ds(base, CE)], eat_vs[b], in_sems[b])
        pltpu.async_copy(el.at[pl.ds(base, CE)], el_vs[b], in_sems[b])

    def wait_in(b):
        pltpu.make_async_copy(
            eat.at[:, pl.ds(0, CE)], eat_vs[b], in_sems[b]).wait()
        pltpu.make_async_copy(
            el.at[pl.ds(0, CE)], el_vs[b], in_sems[b]).wait()

    def start_out(c, b):
        tb = c * NT
        pltpu.async_copy(
            out_as[b], edge_out.at[0, pl.ds(tb, NT), :, :], out_sems[b])
        pltpu.async_copy(
            out_bs[b], edge_out.at[1, pl.ds(tb, NT), pl.ds(0, 2), :],
            out_sems[b])

    def wait_out(b):
        pltpu.make_async_copy(
            out_as[b], edge_out.at[0, pl.ds(0, NT), :, :], out_sems[b]).wait()
        pltpu.make_async_copy(
            out_bs[b], edge_out.at[1, pl.ds(0, NT), pl.ds(0, 2), :],
            out_sems[b]).wait()

    def compute(b):
        eat_v, el_v, out_a, out_b = eat_vs[b], el_vs[b], out_as[b], out_bs[b]

        @plsc.parallel_loop(0, CE // 16, unroll=4)
        def _(i):
            t = i >> 3
            sub = (i & 7) * 16
            sl = pl.ds(i * 16, 16)
            ti = eat_v[0, sl]
            tj = eat_v[1, sl]
            r = el_v[sl]
            idx = ti * 2 + tj
            bits = lax.bitcast_convert_type(r, _i32)
            ef = ((bits >> 23) - 127).astype(_f32)
            m = lax.bitcast_convert_type(
                (bits & 0x007FFFFF) | 0x3F800000, _f32)
            lnm = -1.7306317 + m * (2.7922552 + m * (
                -1.442481 + m * (0.43586185 + m * -0.054862853)))
            lnr = ef * LN2 + lnm
            for_ = 1.0 / (r * (1.0 + jnp.exp((r - RS) * INV_W)))
            dsub = pl.ds(sub, 16)
            for k in range(10):
                w = plsc.load_gather(tab_v, [idx + (4 * k)])
                na2 = lax.bitcast_convert_type(w, _f32)
                cval = lax.bitcast_convert_type(w << 16, _f32)
                v = cval * jnp.exp(na2 * lnr) * for_
                if k < 8:
                    out_a[t, k, dsub] = v
                else:
                    out_b[t, k - 8, dsub] = v

    start_in(wid, 0)

    def pair_body(p, carry):
        for h in range(2):
            s = 2 * p + h
            c = wid + s * NW

            @pl.when(c < NCH)
            def _(c=c, h=h, s=s, p=p):
                cn = wid + (s + 1) * NW

                @pl.when(cn < NCH)
                def _():
                    start_in(cn, 1 - h)

                wait_in(h)

                @pl.when(p > 0)
                def _():
                    wait_out(h)

                compute(h)
                start_out(c, h)

        return carry

    lax.fori_loop(0, SMAX // 2, pair_body, 0)
    wait_out(0)
    wait_out(1)

    @pl.when(wid < NODE_WORKERS)
    def _():
        nbase = wid * NPW
        pltpu.sync_copy(at.at[pl.ds(nbase, NPW)], at_v)

        @plsc.parallel_loop(0, NPW // 16, unroll=4)
        def _(i):
            a = at_v[pl.ds(i * 16, 16)]
            nids = lane + i * 16
            for j in range(3):
                w = plsc.load_gather(tab_v, [a + (40 + 2 * j)])
                plsc.store_scatter(
                    outn_v, [nids, jnp.full((16,), j, _i32)],
                    lax.bitcast_convert_type(w, _f32))

        pltpu.sync_copy(outn_v, node_out.at[pl.ds(nbase, NPW), :])


_sc_kernel = functools.partial(
    pl.kernel,
    _sc_body,
    out_type=(
        jax.ShapeDtypeStruct((2, N_EDGES // 128, 8, 128), jnp.float32),
        jax.ShapeDtypeStruct((N_NODES, 3), jnp.float32),
    ),
    mesh=plsc.VectorSubcoreMesh(core_axis_name="c", subcore_axis_name="s"),
    compiler_params=pltpu.CompilerParams(
        needs_layout_passes=False, use_tc_tiling_on_sc=False),
    scratch_types=[
        pltpu.VMEM((2, CE), jnp.int32),
        pltpu.VMEM((2, CE), jnp.int32),
        pltpu.VMEM((CE,), jnp.float32),
        pltpu.VMEM((CE,), jnp.float32),
        pltpu.VMEM((NT, 8, 128), jnp.float32),
        pltpu.VMEM((NT, 8, 128), jnp.float32),
        pltpu.VMEM((NT, 2, 128), jnp.float32),
        pltpu.VMEM((NT, 2, 128), jnp.float32),
        pltpu.VMEM((NPW,), jnp.int32),
        pltpu.VMEM((NPW, 3), jnp.float32),
        pltpu.VMEM((64,), jnp.int32),
        pltpu.SemaphoreType.DMA,
        pltpu.SemaphoreType.DMA,
        pltpu.SemaphoreType.DMA,
        pltpu.SemaphoreType.DMA,
    ],
)()


def kernel(edge_atom_types, edge_length, atom_type, hopping_param,
           onsite_param, bond_length_table):
    refl = hopping_param[jnp.array(_REFLECTIVE, dtype=_i32)]
    mask = jnp.zeros((10,), dtype=hopping_param.dtype)
    mask = mask.at[jnp.array(_SAME_ORB, dtype=_i32)].set(1.0)[None, :, None]
    hp = hopping_param * (1.0 - mask) + 0.5 * (hopping_param + refl) * mask

    alpha1 = hp[..., 0]
    a2 = jnp.abs(hp[..., 1])
    ti_b = jnp.array([0, 0, 1, 1], dtype=_i32)
    tj_b = jnp.array([0, 1, 0, 1], dtype=_i32)
    r0b = 0.5 * (bond_length_table[ti_b] + bond_length_table[tj_b]) / BOHR
    cb = alpha1 * r0b[:, None] ** (1.0 + a2)
    node_t = onsite_param[:, :, 0]

    cb_bits = lax.bitcast_convert_type(
        cb.astype(jnp.bfloat16), jnp.uint16).astype(_i32)
    na2_bits = lax.bitcast_convert_type(
        (-a2).astype(jnp.bfloat16), jnp.uint16).astype(_i32)
    packed = (na2_bits << 16) | cb_bits

    tab = jnp.zeros((64,), dtype=_i32)
    tab = tab.at[0:40].set(packed.T.reshape(-1))
    tab = tab.at[40:46].set(lax.bitcast_convert_type(
        node_t.astype(_f32), _i32).T.reshape(-1))
    edge_tiles, node_features = _sc_kernel(
        edge_atom_types.astype(_i32), edge_length,
        atom_type.astype(_i32), tab)
    edge_features = edge_tiles.transpose(1, 3, 0, 2).reshape(
        N_EDGES, 16)[:, :10]
    return edge_features, node_features
